# Initial kernel scaffold; baseline (speedup 1.0000x reference)
import jax
import jax.numpy as jnp
from jax.experimental import pallas as pl


def kernel(fiber_idx, s):
    raise NotImplementedError("write your pallas kernel here")


# speedup vs baseline: 2.9632x; 2.9632x over previous
import functools

import jax
import jax.numpy as jnp
from jax import lax
from jax.experimental import pallas as pl
from jax.experimental.pallas import tpu as pltpu
from jax.experimental.pallas import tpu_sc as plsc

NC = 2
NS = 16
NW = NC * NS
CH = 128
D = 128


@functools.partial(jax.jit, static_argnames=("n_ch",))
def _sc_gather(idx, table, n_ch):
    mesh = plsc.VectorSubcoreMesh(core_axis_name="c", subcore_axis_name="s")

    @functools.partial(
        pl.kernel,
        out_type=jax.ShapeDtypeStruct((NW, n_ch, CH, D), jnp.float32),
        mesh=mesh,
        scratch_types=[
            pltpu.VMEM((n_ch, CH), jnp.int32),
            pltpu.VMEM((CH, D), jnp.float32),
            pltpu.SemaphoreType.DMA,
        ],
    )
    def k(idx_hbm, table_hbm, out_hbm, idx_v, rows_v, gsem):
        wid = lax.axis_index("s") * NC + lax.axis_index("c")
        pltpu.sync_copy(idx_hbm.at[wid], idx_v)

        def body(c, carry):
            pltpu.async_copy(table_hbm.at[idx_v.at[c]], rows_v, gsem).wait()
            pltpu.sync_copy(rows_v, out_hbm.at[wid, c])
            return carry

        lax.fori_loop(0, n_ch, body, 0, unroll=False)

    return k(idx, table)


def kernel(fiber_idx, s):
    B, K = fiber_idx.shape
    total = B * K
    n_ch = total // (NW * CH)
    idx = fiber_idx.reshape(NW, n_ch, CH).astype(jnp.int32)
    out = _sc_gather(idx, s, n_ch)
    return out.reshape(B, K, s.shape[1])


# speedup vs baseline: 3.3392x; 1.1269x over previous
import functools

import jax
import jax.numpy as jnp
from jax import lax
from jax.experimental import pallas as pl
from jax.experimental.pallas import tpu as pltpu
from jax.experimental.pallas import tpu_sc as plsc

NC = 2
NS = 16
NW = NC * NS
CH = 128
G = 2
D = 128


@functools.partial(jax.jit, static_argnames=("n_ch",))
def _sc_gather(idx, table, n_ch):
    ngroups = n_ch // G
    mesh = plsc.VectorSubcoreMesh(core_axis_name="c", subcore_axis_name="s")

    @functools.partial(
        pl.kernel,
        out_type=jax.ShapeDtypeStruct((NW, n_ch, CH, D), jnp.float32),
        mesh=mesh,
        scratch_types=[
            pltpu.VMEM((n_ch, CH), jnp.int32),
            pltpu.VMEM((G, CH, D), jnp.float32),
            pltpu.VMEM((G, CH, D), jnp.float32),
            pltpu.SemaphoreType.DMA,
            pltpu.SemaphoreType.DMA,
            pltpu.SemaphoreType.DMA,
            pltpu.SemaphoreType.DMA,
        ],
    )
    def k(idx_hbm, table_hbm, out_hbm, idx_v, rows0, rows1, gsem0, gsem1,
          osem0, osem1):
        wid = lax.axis_index("s") * NC + lax.axis_index("c")
        pltpu.sync_copy(idx_hbm.at[wid], idx_v)
        rows = (rows0, rows1)
        gsem = (gsem0, gsem1)
        osem = (osem0, osem1)

        def body(g, carry):
            for b in range(2):
                is_b = (g % 2) == b
                pb = 1 - b

                @pl.when(is_b & (g < ngroups))
                def _():
                    @pl.when(g >= 2)
                    def _():
                        pltpu.make_async_copy(
                            out_hbm.at[wid, pl.ds((g - 2) * G, G)], rows[b],
                            osem[b]).wait()
                    for j in range(G):
                        pltpu.async_copy(table_hbm.at[idx_v.at[g * G + j]],
                                         rows[b].at[j], gsem[b])

                @pl.when(is_b & (g >= 1) & (g <= ngroups))
                def _():
                    pltpu.make_async_copy(
                        out_hbm.at[wid, pl.ds((g - 1) * G, G)], rows[pb],
                        gsem[pb]).wait()
                    pltpu.async_copy(rows[pb],
                                     out_hbm.at[wid, pl.ds((g - 1) * G, G)],
                                     osem[pb])
            return carry

        lax.fori_loop(0, ngroups + 1, body, 0, unroll=False)

        for b in range(2):
            pltpu.make_async_copy(out_hbm.at[wid, pl.ds(0, G)], rows[b],
                                  osem[b]).wait()

    return k(idx, table)


def kernel(fiber_idx, s):
    B, K = fiber_idx.shape
    total = B * K
    n_ch = total // (NW * CH)
    idx = fiber_idx.reshape(NW, n_ch, CH).astype(jnp.int32)
    out = _sc_gather(idx, s, n_ch)
    return out.reshape(B, K, s.shape[1])


# speedup vs baseline: 5.9485x; 1.7814x over previous
import functools

import jax
import jax.numpy as jnp
from jax import lax
from jax.experimental import pallas as pl
from jax.experimental.pallas import tpu as pltpu
from jax.experimental.pallas import tpu_sc as plsc

NC = 2
NS = 16
NW = NC * NS
BB = 8


@jax.jit
def _sc_gather(idx, table):
    B, K = idx.shape
    _, D = table.shape
    b_per_w = B // NW
    ngroups = b_per_w // BB
    mesh = plsc.VectorSubcoreMesh(core_axis_name="c", subcore_axis_name="s")

    @functools.partial(
        pl.kernel,
        out_type=jax.ShapeDtypeStruct((B, K, D), jnp.float32),
        mesh=mesh,
        scratch_types=[
            pltpu.VMEM((b_per_w, K), jnp.int32),
            pltpu.VMEM((BB, K, D), jnp.float32),
            pltpu.VMEM((BB, K, D), jnp.float32),
            pltpu.SemaphoreType.DMA,
            pltpu.SemaphoreType.DMA,
            pltpu.SemaphoreType.DMA,
            pltpu.SemaphoreType.DMA,
        ],
    )
    def k(idx_hbm, table_hbm, out_hbm, idx_v, rows0, rows1, gsem0, gsem1,
          osem0, osem1):
        wid = lax.axis_index("s") * NC + lax.axis_index("c")
        wb = wid * b_per_w
        pltpu.sync_copy(idx_hbm.at[pl.ds(wb, b_per_w)], idx_v)
        rows = (rows0, rows1)
        gsem = (gsem0, gsem1)
        osem = (osem0, osem1)

        def body(g, carry):
            for b in range(2):
                is_b = (g % 2) == b
                pb = 1 - b

                @pl.when(is_b & (g < ngroups))
                def _():
                    @pl.when(g >= 2)
                    def _():
                        pltpu.make_async_copy(
                            out_hbm.at[pl.ds(wb + (g - 2) * BB, BB)], rows[b],
                            osem[b]).wait()
                    for j in range(BB):
                        pltpu.async_copy(
                            table_hbm.at[idx_v.at[g * BB + j]],
                            rows[b].at[j], gsem[b])

                @pl.when(is_b & (g >= 1) & (g <= ngroups))
                def _():
                    pltpu.make_async_copy(
                        out_hbm.at[pl.ds(wb + (g - 1) * BB, BB)], rows[pb],
                        gsem[pb]).wait()
                    pltpu.async_copy(rows[pb],
                                     out_hbm.at[pl.ds(wb + (g - 1) * BB, BB)],
                                     osem[pb])
            return carry

        lax.fori_loop(0, ngroups + 1, body, 0, unroll=False)

        for b in range(2):
            pltpu.make_async_copy(out_hbm.at[pl.ds(wb, BB)], rows[b],
                                  osem[b]).wait()

    return k(idx, table)


def kernel(fiber_idx, s):
    return _sc_gather(fiber_idx.astype(jnp.int32), s)


# speedup vs baseline: 5.9646x; 1.0027x over previous
import functools

import jax
import jax.numpy as jnp
from jax import lax
from jax.experimental import pallas as pl
from jax.experimental.pallas import tpu as pltpu
from jax.experimental.pallas import tpu_sc as plsc

NC = 2
NS = 16
NW = NC * NS
BB = 8


@jax.jit
def _sc_gather(idx, table):
    B, K = idx.shape
    _, D = table.shape
    b_per_w = B // NW
    ngroups = b_per_w // BB
    mesh = plsc.VectorSubcoreMesh(core_axis_name="c", subcore_axis_name="s")

    @functools.partial(
        pl.kernel,
        out_type=jax.ShapeDtypeStruct((B, K, D), jnp.float32),
        mesh=mesh,
        compiler_params=pltpu.CompilerParams(use_tc_tiling_on_sc=True),
        scratch_types=[
            pltpu.VMEM((b_per_w, K), jnp.int32),
            pltpu.VMEM((BB, K, D), jnp.float32),
            pltpu.VMEM((BB, K, D), jnp.float32),
            pltpu.SemaphoreType.DMA,
            pltpu.SemaphoreType.DMA,
            pltpu.SemaphoreType.DMA,
            pltpu.SemaphoreType.DMA,
        ],
    )
    def k(idx_hbm, table_hbm, out_hbm, idx_v, rows0, rows1, gsem0, gsem1,
          osem0, osem1):
        wid = lax.axis_index("s") * NC + lax.axis_index("c")
        wb = wid * b_per_w
        pltpu.sync_copy(idx_hbm.at[pl.ds(wb, b_per_w)], idx_v)
        rows = (rows0, rows1)
        gsem = (gsem0, gsem1)
        osem = (osem0, osem1)

        def body(g, carry):
            for b in range(2):
                is_b = (g % 2) == b
                pb = 1 - b

                @pl.when(is_b & (g < ngroups))
                def _():
                    @pl.when(g >= 2)
                    def _():
                        pltpu.make_async_copy(
                            out_hbm.at[pl.ds(wb + (g - 2) * BB, BB)], rows[b],
                            osem[b]).wait()
                    for j in range(BB):
                        pltpu.async_copy(
                            table_hbm.at[idx_v.at[g * BB + j]],
                            rows[b].at[j], gsem[b])

                @pl.when(is_b & (g >= 1) & (g <= ngroups))
                def _():
                    pltpu.make_async_copy(
                        out_hbm.at[pl.ds(wb + (g - 1) * BB, BB)], rows[pb],
                        gsem[pb]).wait()
                    pltpu.async_copy(rows[pb],
                                     out_hbm.at[pl.ds(wb + (g - 1) * BB, BB)],
                                     osem[pb])
            return carry

        lax.fori_loop(0, ngroups + 1, body, 0, unroll=False)

        for b in range(2):
            pltpu.make_async_copy(out_hbm.at[pl.ds(wb, BB)], rows[b],
                                  osem[b]).wait()

    return k(idx, table)


def kernel(fiber_idx, s):
    return _sc_gather(fiber_idx.astype(jnp.int32), s)


# speedup vs baseline: 10.5660x; 1.7714x over previous
import functools

import jax
import jax.numpy as jnp
from jax import lax
from jax.experimental import pallas as pl
from jax.experimental.pallas import tpu as pltpu
from jax.experimental.pallas import tpu_sc as plsc

NC = 2
NS = 16
NW = NC * NS
NBUF = 4


@jax.jit
def _sc_gather(idx, table):
    W, K, CB = idx.shape
    _, D = table.shape
    B = W * CB
    mesh = plsc.VectorSubcoreMesh(core_axis_name="c", subcore_axis_name="s")

    @functools.partial(
        pl.kernel,
        out_type=jax.ShapeDtypeStruct((K, B, D), jnp.float32),
        mesh=mesh,
        compiler_params=pltpu.CompilerParams(use_tc_tiling_on_sc=True),
        scratch_types=[
            pltpu.VMEM((K, CB), jnp.int32),
            [pltpu.VMEM((CB, D), jnp.float32)] * NBUF,
            [pltpu.SemaphoreType.DMA] * NBUF,
            [pltpu.SemaphoreType.DMA] * NBUF,
        ],
    )
    def kern(idx_hbm, table_hbm, out_hbm, idx_v, rows, gsem, osem):
        wid = lax.axis_index("s") * NC + lax.axis_index("c")
        wb = wid * CB
        pltpu.sync_copy(idx_hbm.at[wid], idx_v)

        def body(k, carry):
            for b in range(NBUF):
                @pl.when(((k % NBUF) == b) & (k < K))
                def _():
                    @pl.when(k >= NBUF)
                    def _():
                        pltpu.make_async_copy(
                            out_hbm.at[k - NBUF, pl.ds(wb, CB)], rows[b],
                            osem[b]).wait()
                    pltpu.async_copy(table_hbm.at[idx_v.at[k]], rows[b],
                                     gsem[b])
            for b in range(NBUF):
                @pl.when((((k - 1) % NBUF) == b) & (k >= 1) & (k <= K))
                def _():
                    pltpu.make_async_copy(
                        out_hbm.at[k - 1, pl.ds(wb, CB)], rows[b],
                        gsem[b]).wait()
                    pltpu.async_copy(rows[b],
                                     out_hbm.at[k - 1, pl.ds(wb, CB)],
                                     osem[b])
            return carry

        lax.fori_loop(0, K + 1, body, 0, unroll=False)

        for b in range(NBUF):
            pltpu.make_async_copy(out_hbm.at[0, pl.ds(wb, CB)], rows[b],
                                  osem[b]).wait()

    return kern(idx, table)


def kernel(fiber_idx, s):
    B, K = fiber_idx.shape
    CB = B // NW
    idx = fiber_idx.astype(jnp.int32).T.reshape(K, NW, CB).transpose(1, 0, 2)
    out_km = _sc_gather(idx, s)
    return out_km.transpose(1, 0, 2)
